# baseline (speedup 1.0000x reference)
import math

import jax
import jax.numpy as jnp
from jax import lax
from jax.experimental import pallas as pl
from jax.experimental.pallas import tpu as pltpu

_SIZE = 100000
_N = 2048
_SMOOTHING = 0.1
_CONF = 1.0 - _SMOOTHING
_S = _SMOOTHING / (_SIZE - 2)
_C0 = (_SIZE - 2) * _S * math.log(_S) + _CONF * math.log(_CONF)

_BR = 512
_BC = 2048
_NCB = (_SIZE + _BC - 1) // _BC


def _tc_body(x_ref, t_ref, out_ref):
    i = pl.program_id(0)
    j = pl.program_id(1)

    @pl.when(jnp.logical_and(i == 0, j == 0))
    def _init():
        out_ref[0, 0] = 0.0

    t = t_ref[...]
    rowmask = t != 0
    xb = x_ref[...]
    cols = lax.broadcasted_iota(jnp.int32, (_BR, _BC), 1) + j * _BC

    hit = (cols == t) & rowmask
    out_ref[0, 0] += -(_CONF - _S) * jnp.sum(jnp.where(hit, xb, 0.0))

    valid = rowmask & (cols < _SIZE)
    out_ref[0, 0] += -_S * jnp.sum(jnp.where(valid, xb, 0.0))

    @pl.when(j == 0)
    def _row_terms():
        x0 = xb[:, 0:1]
        out_ref[0, 0] += _S * jnp.sum(jnp.where(rowmask, x0, 0.0))
        out_ref[0, 0] += _C0 * jnp.sum(jnp.where(rowmask, 1.0, 0.0))


_tc_loss = pl.pallas_call(
    _tc_body,
    grid=(_N // _BR, _NCB),
    in_specs=[
        pl.BlockSpec((_BR, _BC), lambda i, j: (i, j)),
        pl.BlockSpec((_BR, 1), lambda i, j: (i, 0)),
    ],
    out_specs=pl.BlockSpec((1, 1), lambda i, j: (0, 0), memory_space=pltpu.SMEM),
    out_shape=jax.ShapeDtypeStruct((1, 1), jnp.float32),
    compiler_params=pltpu.CompilerParams(
        dimension_semantics=("arbitrary", "arbitrary"),
    ),
)


def kernel(x, target):
    tgt = target.astype(jnp.int32)
    return _tc_loss(x, tgt.reshape(_N, 1))[0, 0]
